# Initial kernel scaffold; baseline (speedup 1.0000x reference)
import jax
import jax.numpy as jnp
from jax.experimental import pallas as pl


def kernel(node_scalars, edge_feats, d, W1, b1, W2, b2, ln_gamma, ln_beta, edge_index):
    raise NotImplementedError("write your pallas kernel here")


# speedup vs baseline: 2.5422x; 2.5422x over previous
import functools

import jax
import jax.numpy as jnp
from jax import lax
from jax.experimental import pallas as pl
from jax.experimental.pallas import tpu as pltpu
from jax.experimental.pallas import tpu_sc as plsc

_N_NODES = 10000
_N_EDGES = 320000
_D = 128
_RBF = 16

_NC = 2
_NS = 16
_NW = _NC * _NS
_CHUNK = 128
_CPW = 79
_EPW = _CPW * _CHUNK
_E_PAD = _NW * _EPW

_BE = 2560


def _node_proj_body(ns_ref, ws_ref, wd_ref, ps_ref, pd_ref):
    x = ns_ref[...]
    ps_ref[...] = jnp.dot(x, ws_ref[...], preferred_element_type=jnp.float32)
    pd_ref[...] = jnp.dot(x, wd_ref[...], preferred_element_type=jnp.float32)


def _gather_add_body(ps_hbm, pd_hbm, src_hbm, dst_hbm, out_hbm,
                     idx_s, idx_d, r1, r2, sem1, sem2):
    wid = lax.axis_index("s") * _NC + lax.axis_index("c")
    base0 = wid * _EPW

    @pl.loop(0, _CPW)
    def _chunk(t):
        base = base0 + t * _CHUNK
        pltpu.sync_copy(src_hbm.at[pl.ds(base, _CHUNK)], idx_s)
        pltpu.sync_copy(dst_hbm.at[pl.ds(base, _CHUNK)], idx_d)
        cp1 = pltpu.async_copy(ps_hbm.at[idx_s], r1, sem1)
        cp2 = pltpu.async_copy(pd_hbm.at[idx_d], r2, sem2)
        cp1.wait()
        cp2.wait()

        @pl.loop(0, _CHUNK)
        def _row(i):
            for j in range(8):
                s = pl.ds(j * 16, 16)
                r1[i, s] = r1[i, s] + r2[i, s]

        pltpu.sync_copy(r1, out_hbm.at[pl.ds(base, _CHUNK)])


def _mlp_body(g_ref, ef_ref, d_ref, we_ref, wd_ref, w2_ref,
              b1_ref, b2_ref, gam_ref, bet_ref, o_ref):
    ef = ef_ref[...]
    x = g_ref[...] + jnp.dot(ef, we_ref[...], preferred_element_type=jnp.float32)
    x = x + jnp.dot(d_ref[...], wd_ref[...], preferred_element_type=jnp.float32)
    x = x + b1_ref[...]
    x = x * (1.0 / (1.0 + jnp.exp(-x)))
    y = jnp.dot(x, w2_ref[...], preferred_element_type=jnp.float32) + b2_ref[...]
    y = y * (1.0 / (1.0 + jnp.exp(-y)))
    z = ef + y
    mu = jnp.mean(z, axis=1, keepdims=True)
    zc = z - mu
    var = jnp.mean(zc * zc, axis=1, keepdims=True)
    o_ref[...] = zc * lax.rsqrt(var + 1e-5) * gam_ref[...] + bet_ref[...]


def kernel(node_scalars, edge_feats, d, W1, b1, W2, b2, ln_gamma, ln_beta, edge_index):
    idx = edge_index.astype(jnp.int32)
    src = jnp.pad(idx[0], (0, _E_PAD - _N_EDGES))
    dst = jnp.pad(idx[1], (0, _E_PAD - _N_EDGES))

    ps, pd = pl.pallas_call(
        _node_proj_body,
        out_shape=[jax.ShapeDtypeStruct((_N_NODES, _D), jnp.float32)] * 2,
    )(node_scalars, W1[0:_D], W1[_D:2 * _D])

    sc_gather = pl.kernel(
        _gather_add_body,
        out_type=jax.ShapeDtypeStruct((_E_PAD, _D), jnp.float32),
        mesh=plsc.VectorSubcoreMesh(core_axis_name="c", subcore_axis_name="s"),
        scratch_types=[
            pltpu.VMEM((_CHUNK,), jnp.int32),
            pltpu.VMEM((_CHUNK,), jnp.int32),
            pltpu.VMEM((_CHUNK, _D), jnp.float32),
            pltpu.VMEM((_CHUNK, _D), jnp.float32),
            pltpu.SemaphoreType.DMA,
            pltpu.SemaphoreType.DMA,
        ],
    )
    g = sc_gather(ps, pd, src, dst)

    full = lambda i: (0, 0)
    out = pl.pallas_call(
        _mlp_body,
        grid=(_N_EDGES // _BE,),
        in_specs=[
            pl.BlockSpec((_BE, _D), lambda i: (i, 0)),
            pl.BlockSpec((_BE, _D), lambda i: (i, 0)),
            pl.BlockSpec((_BE, _RBF), lambda i: (i, 0)),
            pl.BlockSpec((_D, _D), full),
            pl.BlockSpec((_RBF, _D), full),
            pl.BlockSpec((_D, _D), full),
            pl.BlockSpec((1, _D), full),
            pl.BlockSpec((1, _D), full),
            pl.BlockSpec((1, _D), full),
            pl.BlockSpec((1, _D), full),
        ],
        out_specs=pl.BlockSpec((_BE, _D), lambda i: (i, 0)),
        out_shape=jax.ShapeDtypeStruct((_N_EDGES, _D), jnp.float32),
    )(g, edge_feats, d, W1[2 * _D:3 * _D], W1[3 * _D:], W2,
      b1.reshape(1, _D), b2.reshape(1, _D),
      ln_gamma.reshape(1, _D), ln_beta.reshape(1, _D))
    return out
